# baseline (speedup 1.0000x reference)
import functools

import jax
import jax.numpy as jnp
from jax import lax
from jax.experimental import pallas as pl
from jax.experimental.pallas import tpu as pltpu
from jax.experimental.pallas import tpu_sc as plsc

_NC = 2
_NS = 16
_NW = _NC * _NS
_CH = 80


def _spmm_sc(m, src2, dst2, w2, n_nodes):
    nf, _, d = m.shape
    n_chunks, ch = src2.shape
    cpw = n_chunks // _NW
    rpt = n_nodes // _NS
    zb = 25
    mesh = plsc.VectorSubcoreMesh(core_axis_name="c", subcore_axis_name="s")

    @functools.partial(
        pl.kernel,
        out_type=jax.ShapeDtypeStruct((_NC, nf, n_nodes, d), jnp.float32),
        mesh=mesh,
        scratch_types=[
            pltpu.VMEM((cpw, ch), jnp.int32),
            pltpu.VMEM((cpw, ch), jnp.int32),
            pltpu.VMEM((cpw, ch), jnp.float32),
            pltpu.VMEM((2, ch, d), jnp.float32),
            pltpu.VMEM((zb, d), jnp.float32),
            pltpu.VMEM_SHARED((n_nodes, d), jnp.float32),
            pltpu.SemaphoreType.DMA,
        ],
        compiler_params=pltpu.CompilerParams(use_tc_tiling_on_sc=False,
                                             needs_layout_passes=False),
    )
    def spmm(m_hbm, src_hbm, dst_hbm, w_hbm, out_hbm,
             src_v, dst_v, w_v, rows_v, zb_v, acc_sh, gsem):
        cid = lax.axis_index("c")
        sid = lax.axis_index("s")
        wid = cid * _NS + sid
        row0 = wid * cpw

        pltpu.sync_copy(src_hbm.at[pl.ds(row0, cpw)], src_v)
        pltpu.sync_copy(dst_hbm.at[pl.ds(row0, cpw)], dst_v)
        pltpu.sync_copy(w_hbm.at[pl.ds(row0, cpw)], w_v)

        tail = d % 16
        lane = lax.iota(jnp.int32, 16)
        tmask = lane < tail
        tcol = (d - tail) + lane

        zeros16 = jnp.zeros((16,), jnp.float32)
        for r in range(zb):
            for j in range(d // 16):
                zb_v[r, pl.ds(j * 16, 16)] = zeros16
            if tail:
                plsc.store_scatter(zb_v, [jnp.full((16,), r, jnp.int32), tcol],
                                   zeros16, mask=tmask)
        acc0 = sid * rpt

        for fh in range(nf):
            m_f = m_hbm.at[fh]

            for i in range(rpt // zb):
                pltpu.sync_copy(zb_v, acc_sh.at[pl.ds(acc0 + i * zb, zb)])
            plsc.subcore_barrier()

            def gather_start(c, buf):
                pltpu.async_copy(m_f.at[src_v.at[c]], rows_v.at[buf], gsem)

            def gather_wait(c, buf):
                pltpu.make_async_copy(m_f.at[src_v.at[c]], rows_v.at[buf],
                                      gsem).wait()

            def process(c, buf):
                c16 = jnp.full((16,), c, jnp.int32)

                @pl.loop(0, ch, unroll=4)
                def edge(e):
                    e16 = jnp.full((16,), e, jnp.int32)
                    ws = plsc.load_gather(w_v, [c16, e16])
                    for j in range(d // 16):
                        sl = pl.ds(j * 16, 16)
                        rows_v[buf, e, sl] = rows_v[buf, e, sl] * ws
                    if tail:
                        tv = plsc.load_gather(rows_v.at[buf], [e16, tcol],
                                              mask=tmask)
                        plsc.store_scatter(rows_v.at[buf], [e16, tcol],
                                           tv * ws, mask=tmask)
                pltpu.sync_copy(rows_v.at[buf], acc_sh.at[dst_v.at[c]],
                                add=True)

            gather_start(0, 0)

            def pair(g, _):
                c = g * 2
                gather_start(c + 1, 1)
                gather_wait(c, 0)
                process(c, 0)
                gather_start(c + 2, 0)
                gather_wait(c + 1, 1)
                process(c + 1, 1)
                return 0

            lax.fori_loop(0, (cpw - 1) // 2, pair, 0)
            gather_wait(cpw - 1, 0)
            process(cpw - 1, 0)

            plsc.subcore_barrier()
            pltpu.sync_copy(acc_sh.at[pl.ds(acc0, rpt)],
                            out_hbm.at[cid, fh, pl.ds(acc0, rpt)])
            plsc.subcore_barrier()

    return spmm(m, src2, dst2, w2)


def _mm1_kernel(a_ref, b_ref, o_ref):
    a = a_ref[...]
    o_ref[0] = jnp.dot(a, b_ref[0], preferred_element_type=jnp.float32)
    o_ref[1] = jnp.dot(a, b_ref[1], preferred_element_type=jnp.float32)


def _matmul1(a, b2, bm=400):
    m, k = a.shape
    n = b2.shape[2]
    return pl.pallas_call(
        _mm1_kernel,
        grid=(m // bm,),
        in_specs=[
            pl.BlockSpec((bm, k), lambda i: (i, 0)),
            pl.BlockSpec((2, k, n), lambda i: (0, 0, 0)),
        ],
        out_specs=pl.BlockSpec((2, bm, n), lambda i: (0, i, 0)),
        out_shape=jax.ShapeDtypeStruct((2, m, n), jnp.float32),
    )(a, b2)


def _fuse2_kernel(p_ref, w_ref, h_ref, o_ref):
    hblk = jnp.concatenate(
        [jnp.maximum(p_ref[0, 0] + p_ref[1, 0], 0.0),
         jnp.maximum(p_ref[0, 1] + p_ref[1, 1], 0.0)], axis=-1)
    h_ref[...] = hblk
    o_ref[...] = jnp.dot(hblk, w_ref[...], preferred_element_type=jnp.float32)


def _fuse2(p1, w2, bm=400):
    _, _, m, k2 = p1.shape
    n = w2.shape[1]
    return pl.pallas_call(
        _fuse2_kernel,
        grid=(m // bm,),
        in_specs=[
            pl.BlockSpec((2, 2, bm, k2), lambda i: (0, 0, i, 0)),
            pl.BlockSpec((2 * k2, n), lambda i: (0, 0)),
        ],
        out_specs=[
            pl.BlockSpec((bm, 2 * k2), lambda i: (i, 0)),
            pl.BlockSpec((bm, n), lambda i: (i, 0)),
        ],
        out_shape=[
            jax.ShapeDtypeStruct((m, 2 * k2), jnp.float32),
            jax.ShapeDtypeStruct((m, n), jnp.float32),
        ],
    )(p1, w2)


def _comb_kernel(p_ref, o_ref):
    o_ref[...] = p_ref[0, 0] + p_ref[1, 0]


def _combine(p2, bm=400):
    _, _, m, n = p2.shape
    return pl.pallas_call(
        _comb_kernel,
        grid=(m // bm,),
        in_specs=[pl.BlockSpec((2, 1, bm, n), lambda i: (0, 0, i, 0))],
        out_specs=pl.BlockSpec((bm, n), lambda i: (i, 0)),
        out_shape=jax.ShapeDtypeStruct((m, n), jnp.float32),
    )(p2)


def kernel(x, edge_index, edge_weight, W1, W2):
    n = x.shape[0]
    d_hid = W1.shape[1]
    src2 = edge_index[0].reshape(-1, _CH)
    dst2 = edge_index[1].reshape(-1, _CH)
    w2e = edge_weight.reshape(-1, _CH)
    w1s = W1.T.reshape(2, d_hid // 2, W1.shape[0]).transpose(0, 2, 1)

    xw1 = _matmul1(x, w1s)
    p1 = _spmm_sc(xw1, src2, dst2, w2e, n)
    h, hw2 = _fuse2(p1, W2)
    p2 = _spmm_sc(hw2[None], src2, dst2, w2e, n)
    out = _combine(p2)
    return (out, h)


# speedup vs baseline: 1.2174x; 1.2174x over previous
import functools

import jax
import jax.numpy as jnp
from jax import lax
from jax.experimental import pallas as pl
from jax.experimental.pallas import tpu as pltpu
from jax.experimental.pallas import tpu_sc as plsc

_NC = 2
_NS = 16
_NW = _NC * _NS
_CH = 80


def _spmm_sc(m, src2, dst2, w2, n_nodes):
    nf, _, d = m.shape
    n_chunks, ch = src2.shape
    cpw = n_chunks // _NW
    rpt = n_nodes // _NS
    zb = 25
    mesh = plsc.VectorSubcoreMesh(core_axis_name="c", subcore_axis_name="s")

    @functools.partial(
        pl.kernel,
        out_type=jax.ShapeDtypeStruct((_NC, nf, n_nodes, d), jnp.float32),
        mesh=mesh,
        scratch_types=[
            pltpu.VMEM((cpw, ch), jnp.int32),
            pltpu.VMEM((cpw, ch), jnp.int32),
            pltpu.VMEM((cpw, ch), jnp.float32),
            pltpu.VMEM((2, ch, d), jnp.float32),
            pltpu.VMEM((2, ch, d), jnp.float32),
            pltpu.VMEM((zb, d), jnp.float32),
            pltpu.VMEM_SHARED((n_nodes, d), jnp.float32),
            pltpu.SemaphoreType.DMA,
        ],
        compiler_params=pltpu.CompilerParams(use_tc_tiling_on_sc=False,
                                             needs_layout_passes=False),
    )
    def spmm(m_hbm, src_hbm, dst_hbm, w_hbm, out_hbm,
             src_v, dst_v, w_v, rows_v, srows_v, zb_v, acc_sh, gsem):
        cid = lax.axis_index("c")
        sid = lax.axis_index("s")
        wid = cid * _NS + sid
        row0 = wid * cpw

        pltpu.sync_copy(src_hbm.at[pl.ds(row0, cpw)], src_v)
        pltpu.sync_copy(dst_hbm.at[pl.ds(row0, cpw)], dst_v)
        pltpu.sync_copy(w_hbm.at[pl.ds(row0, cpw)], w_v)

        tail = d % 16
        lane = lax.iota(jnp.int32, 16)
        tmask = lane < tail
        tcol = (d - tail) + lane

        zeros16 = jnp.zeros((16,), jnp.float32)
        for r in range(zb):
            for j in range(d // 16):
                zb_v[r, pl.ds(j * 16, 16)] = zeros16
            if tail:
                plsc.store_scatter(zb_v, [jnp.full((16,), r, jnp.int32), tcol],
                                   zeros16, mask=tmask)
        acc0 = sid * rpt

        for fh in range(nf):
            m_f = m_hbm.at[fh]

            for i in range(rpt // zb):
                pltpu.sync_copy(zb_v, acc_sh.at[pl.ds(acc0 + i * zb, zb)])
            plsc.subcore_barrier()

            def gather_start(c, buf):
                pltpu.async_copy(m_f.at[src_v.at[c]], rows_v.at[buf], gsem)

            def gather_wait(c, buf):
                pltpu.make_async_copy(m_f.at[src_v.at[c]], rows_v.at[buf],
                                      gsem).wait()

            def process(c, buf):
                c16 = jnp.full((16,), c, jnp.int32)

                @plsc.parallel_loop(0, ch, unroll=4)
                def edge(e):
                    e16 = jnp.full((16,), e, jnp.int32)
                    ws = plsc.load_gather(w_v, [c16, e16])
                    for j in range(d // 16):
                        sl = pl.ds(j * 16, 16)
                        srows_v[buf, e, sl] = rows_v[buf, e, sl] * ws
                    if tail:
                        tv = plsc.load_gather(rows_v.at[buf], [e16, tcol],
                                              mask=tmask)
                        plsc.store_scatter(srows_v.at[buf], [e16, tcol],
                                           tv * ws, mask=tmask)
                pltpu.sync_copy(srows_v.at[buf], acc_sh.at[dst_v.at[c]],
                                add=True)

            gather_start(0, 0)

            def pair(g, _):
                c = g * 2
                gather_start(c + 1, 1)
                gather_wait(c, 0)
                process(c, 0)
                gather_start(c + 2, 0)
                gather_wait(c + 1, 1)
                process(c + 1, 1)
                return 0

            lax.fori_loop(0, (cpw - 1) // 2, pair, 0)
            gather_wait(cpw - 1, 0)
            process(cpw - 1, 0)

            plsc.subcore_barrier()
            pltpu.sync_copy(acc_sh.at[pl.ds(acc0, rpt)],
                            out_hbm.at[cid, fh, pl.ds(acc0, rpt)])
            plsc.subcore_barrier()

    return spmm(m, src2, dst2, w2)


def _mm1_kernel(a_ref, b_ref, o_ref):
    a = a_ref[...]
    o_ref[0] = jnp.dot(a, b_ref[0], preferred_element_type=jnp.float32)
    o_ref[1] = jnp.dot(a, b_ref[1], preferred_element_type=jnp.float32)


def _matmul1(a, b2, bm=400):
    m, k = a.shape
    n = b2.shape[2]
    return pl.pallas_call(
        _mm1_kernel,
        grid=(m // bm,),
        in_specs=[
            pl.BlockSpec((bm, k), lambda i: (i, 0)),
            pl.BlockSpec((2, k, n), lambda i: (0, 0, 0)),
        ],
        out_specs=pl.BlockSpec((2, bm, n), lambda i: (0, i, 0)),
        out_shape=jax.ShapeDtypeStruct((2, m, n), jnp.float32),
    )(a, b2)


def _fuse2_kernel(p_ref, w_ref, h_ref, o_ref):
    hblk = jnp.concatenate(
        [jnp.maximum(p_ref[0, 0] + p_ref[1, 0], 0.0),
         jnp.maximum(p_ref[0, 1] + p_ref[1, 1], 0.0)], axis=-1)
    h_ref[...] = hblk
    o_ref[...] = jnp.dot(hblk, w_ref[...], preferred_element_type=jnp.float32)


def _fuse2(p1, w2, bm=400):
    _, _, m, k2 = p1.shape
    n = w2.shape[1]
    return pl.pallas_call(
        _fuse2_kernel,
        grid=(m // bm,),
        in_specs=[
            pl.BlockSpec((2, 2, bm, k2), lambda i: (0, 0, i, 0)),
            pl.BlockSpec((2 * k2, n), lambda i: (0, 0)),
        ],
        out_specs=[
            pl.BlockSpec((bm, 2 * k2), lambda i: (i, 0)),
            pl.BlockSpec((bm, n), lambda i: (i, 0)),
        ],
        out_shape=[
            jax.ShapeDtypeStruct((m, 2 * k2), jnp.float32),
            jax.ShapeDtypeStruct((m, n), jnp.float32),
        ],
    )(p1, w2)


def _comb_kernel(p_ref, o_ref):
    o_ref[...] = p_ref[0, 0] + p_ref[1, 0]


def _combine(p2, bm=400):
    _, _, m, n = p2.shape
    return pl.pallas_call(
        _comb_kernel,
        grid=(m // bm,),
        in_specs=[pl.BlockSpec((2, 1, bm, n), lambda i: (0, 0, i, 0))],
        out_specs=pl.BlockSpec((bm, n), lambda i: (i, 0)),
        out_shape=jax.ShapeDtypeStruct((m, n), jnp.float32),
    )(p2)


def kernel(x, edge_index, edge_weight, W1, W2):
    n = x.shape[0]
    d_hid = W1.shape[1]
    src2 = edge_index[0].reshape(-1, _CH)
    dst2 = edge_index[1].reshape(-1, _CH)
    w2e = edge_weight.reshape(-1, _CH)
    w1s = W1.T.reshape(2, d_hid // 2, W1.shape[0]).transpose(0, 2, 1)

    xw1 = _matmul1(x, w1s)
    p1 = _spmm_sc(xw1, src2, dst2, w2e, n)
    h, hw2 = _fuse2(p1, W2)
    p2 = _spmm_sc(hw2[None], src2, dst2, w2e, n)
    out = _combine(p2)
    return (out, h)


# speedup vs baseline: 1.3633x; 1.1199x over previous
import functools

import jax
import jax.numpy as jnp
from jax import lax
from jax.experimental import pallas as pl
from jax.experimental.pallas import tpu as pltpu
from jax.experimental.pallas import tpu_sc as plsc

_NC = 2
_NS = 16
_NW = _NC * _NS
_CH = 80


def _spmm_sc(m, src2, dst2, w2, n_nodes):
    nf, _, d = m.shape
    n_chunks, ch = src2.shape
    cpw = n_chunks // _NW
    rpt = n_nodes // _NS
    zb = 25
    mesh = plsc.VectorSubcoreMesh(core_axis_name="c", subcore_axis_name="s")

    @functools.partial(
        pl.kernel,
        out_type=jax.ShapeDtypeStruct((_NC, nf, n_nodes, d), jnp.float32),
        mesh=mesh,
        scratch_types=[
            pltpu.VMEM((cpw, ch), jnp.int32),
            pltpu.VMEM((cpw, ch), jnp.int32),
            pltpu.VMEM((cpw, ch), jnp.float32),
            pltpu.VMEM((2, ch, d), jnp.float32),
            pltpu.VMEM((2, ch, d), jnp.float32),
            pltpu.VMEM((zb, d), jnp.float32),
            pltpu.VMEM_SHARED((n_nodes, d), jnp.float32),
            pltpu.SemaphoreType.DMA,
            pltpu.SemaphoreType.DMA,
        ],
        compiler_params=pltpu.CompilerParams(use_tc_tiling_on_sc=False,
                                             needs_layout_passes=False),
    )
    def spmm(m_hbm, src_hbm, dst_hbm, w_hbm, out_hbm,
             src_v, dst_v, w_v, rows_v, srows_v, zb_v, acc_sh, gsem, ssem):
        cid = lax.axis_index("c")
        sid = lax.axis_index("s")
        wid = cid * _NS + sid
        row0 = wid * cpw

        pltpu.sync_copy(src_hbm.at[pl.ds(row0, cpw)], src_v)
        pltpu.sync_copy(dst_hbm.at[pl.ds(row0, cpw)], dst_v)
        pltpu.sync_copy(w_hbm.at[pl.ds(row0, cpw)], w_v)

        tail = d % 16
        lane = lax.iota(jnp.int32, 16)
        tmask = lane < tail
        tcol = (d - tail) + lane

        zeros16 = jnp.zeros((16,), jnp.float32)
        for r in range(zb):
            for j in range(d // 16):
                zb_v[r, pl.ds(j * 16, 16)] = zeros16
            if tail:
                plsc.store_scatter(zb_v, [jnp.full((16,), r, jnp.int32), tcol],
                                   zeros16, mask=tmask)
        acc0 = sid * rpt

        for fh in range(nf):
            m_f = m_hbm.at[fh]

            for i in range(rpt // zb):
                pltpu.sync_copy(zb_v, acc_sh.at[pl.ds(acc0 + i * zb, zb)])
            plsc.subcore_barrier()

            def gather_start(c, buf):
                pltpu.async_copy(m_f.at[src_v.at[c]], rows_v.at[buf], gsem)

            def gather_wait(c, buf):
                pltpu.make_async_copy(m_f.at[src_v.at[c]], rows_v.at[buf],
                                      gsem).wait()

            def multiply(c, buf):
                c16 = jnp.full((16,), c, jnp.int32)

                @plsc.parallel_loop(0, ch, unroll=4)
                def edge(e):
                    e16 = jnp.full((16,), e, jnp.int32)
                    ws = plsc.load_gather(w_v, [c16, e16])
                    for j in range(d // 16):
                        sl = pl.ds(j * 16, 16)
                        srows_v[buf, e, sl] = rows_v[buf, e, sl] * ws
                    if tail:
                        tv = plsc.load_gather(rows_v.at[buf], [e16, tcol],
                                              mask=tmask)
                        plsc.store_scatter(srows_v.at[buf], [e16, tcol],
                                           tv * ws, mask=tmask)

            def scatter_start(c, buf):
                pltpu.async_copy(srows_v.at[buf], acc_sh.at[dst_v.at[c]],
                                 ssem, add=True)

            def scatter_wait(c, buf):
                pltpu.make_async_copy(srows_v.at[buf], acc_sh.at[dst_v.at[c]],
                                      ssem).wait()

            gather_start(0, 0)

            def pair(g, _):
                c = g * 2
                gather_start(c + 1, 1)
                gather_wait(c, 0)

                @pl.when(g > 0)
                def _():
                    scatter_wait(c - 2, 0)
                multiply(c, 0)
                scatter_start(c, 0)
                gather_start(c + 2, 0)
                gather_wait(c + 1, 1)

                @pl.when(g > 0)
                def _():
                    scatter_wait(c - 1, 1)
                multiply(c + 1, 1)
                scatter_start(c + 1, 1)
                return 0

            lax.fori_loop(0, (cpw - 1) // 2, pair, 0)
            gather_wait(cpw - 1, 0)
            scatter_wait(cpw - 3, 0)
            multiply(cpw - 1, 0)
            scatter_start(cpw - 1, 0)
            scatter_wait(cpw - 2, 1)
            scatter_wait(cpw - 1, 0)

            plsc.subcore_barrier()
            pltpu.sync_copy(acc_sh.at[pl.ds(acc0, rpt)],
                            out_hbm.at[cid, fh, pl.ds(acc0, rpt)])
            plsc.subcore_barrier()

    return spmm(m, src2, dst2, w2)


def _mm1_kernel(a_ref, b_ref, o_ref):
    a = a_ref[...]
    o_ref[0] = jnp.dot(a, b_ref[0], preferred_element_type=jnp.float32)
    o_ref[1] = jnp.dot(a, b_ref[1], preferred_element_type=jnp.float32)


def _matmul1(a, b2, bm=400):
    m, k = a.shape
    n = b2.shape[2]
    return pl.pallas_call(
        _mm1_kernel,
        grid=(m // bm,),
        in_specs=[
            pl.BlockSpec((bm, k), lambda i: (i, 0)),
            pl.BlockSpec((2, k, n), lambda i: (0, 0, 0)),
        ],
        out_specs=pl.BlockSpec((2, bm, n), lambda i: (0, i, 0)),
        out_shape=jax.ShapeDtypeStruct((2, m, n), jnp.float32),
    )(a, b2)


def _fuse2_kernel(p_ref, w_ref, h_ref, o_ref):
    hblk = jnp.concatenate(
        [jnp.maximum(p_ref[0, 0] + p_ref[1, 0], 0.0),
         jnp.maximum(p_ref[0, 1] + p_ref[1, 1], 0.0)], axis=-1)
    h_ref[...] = hblk
    o_ref[...] = jnp.dot(hblk, w_ref[...], preferred_element_type=jnp.float32)


def _fuse2(p1, w2, bm=400):
    _, _, m, k2 = p1.shape
    n = w2.shape[1]
    return pl.pallas_call(
        _fuse2_kernel,
        grid=(m // bm,),
        in_specs=[
            pl.BlockSpec((2, 2, bm, k2), lambda i: (0, 0, i, 0)),
            pl.BlockSpec((2 * k2, n), lambda i: (0, 0)),
        ],
        out_specs=[
            pl.BlockSpec((bm, 2 * k2), lambda i: (i, 0)),
            pl.BlockSpec((bm, n), lambda i: (i, 0)),
        ],
        out_shape=[
            jax.ShapeDtypeStruct((m, 2 * k2), jnp.float32),
            jax.ShapeDtypeStruct((m, n), jnp.float32),
        ],
    )(p1, w2)


def _comb_kernel(p_ref, o_ref):
    o_ref[...] = p_ref[0, 0] + p_ref[1, 0]


def _combine(p2, bm=400):
    _, _, m, n = p2.shape
    return pl.pallas_call(
        _comb_kernel,
        grid=(m // bm,),
        in_specs=[pl.BlockSpec((2, 1, bm, n), lambda i: (0, 0, i, 0))],
        out_specs=pl.BlockSpec((bm, n), lambda i: (i, 0)),
        out_shape=jax.ShapeDtypeStruct((m, n), jnp.float32),
    )(p2)


def kernel(x, edge_index, edge_weight, W1, W2):
    n = x.shape[0]
    d_hid = W1.shape[1]
    src2 = edge_index[0].reshape(-1, _CH)
    dst2 = edge_index[1].reshape(-1, _CH)
    w2e = edge_weight.reshape(-1, _CH)
    w1s = W1.T.reshape(2, d_hid // 2, W1.shape[0]).transpose(0, 2, 1)

    xw1 = _matmul1(x, w1s)
    p1 = _spmm_sc(xw1, src2, dst2, w2e, n)
    h, hw2 = _fuse2(p1, W2)
    p2 = _spmm_sc(hw2[None], src2, dst2, w2e, n)
    out = _combine(p2)
    return (out, h)


# speedup vs baseline: 1.3654x; 1.0015x over previous
import functools

import jax
import jax.numpy as jnp
from jax import lax
from jax.experimental import pallas as pl
from jax.experimental.pallas import tpu as pltpu
from jax.experimental.pallas import tpu_sc as plsc

_NC = 2
_NS = 16
_NW = _NC * _NS
_CH = 80


def _spmm_sc(m, src2, dst2, w2, n_nodes):
    nf, _, d = m.shape
    n_chunks, ch = src2.shape
    cpw = n_chunks // _NW
    rpt = n_nodes // _NS
    zb = 25
    mesh = plsc.VectorSubcoreMesh(core_axis_name="c", subcore_axis_name="s")

    @functools.partial(
        pl.kernel,
        out_type=jax.ShapeDtypeStruct((_NC, nf, n_nodes, d), jnp.float32),
        mesh=mesh,
        scratch_types=[
            pltpu.VMEM((cpw, ch), jnp.int32),
            pltpu.VMEM((cpw, ch), jnp.int32),
            pltpu.VMEM((cpw, ch), jnp.float32),
            pltpu.VMEM((2, ch, d), jnp.float32),
            pltpu.VMEM((2, ch, d), jnp.float32),
            pltpu.VMEM((zb, d), jnp.float32),
            pltpu.VMEM_SHARED((n_nodes, d), jnp.float32),
            pltpu.SemaphoreType.DMA,
            pltpu.SemaphoreType.DMA,
        ],
        compiler_params=pltpu.CompilerParams(use_tc_tiling_on_sc=False,
                                             needs_layout_passes=False),
    )
    def spmm(m_hbm, src_hbm, dst_hbm, w_hbm, out_hbm,
             src_v, dst_v, w_v, rows_v, srows_v, zb_v, acc_sh, gsem, ssem):
        cid = lax.axis_index("c")
        sid = lax.axis_index("s")
        wid = cid * _NS + sid
        row0 = wid * cpw

        pltpu.sync_copy(src_hbm.at[pl.ds(row0, cpw)], src_v)
        pltpu.sync_copy(dst_hbm.at[pl.ds(row0, cpw)], dst_v)
        pltpu.sync_copy(w_hbm.at[pl.ds(row0, cpw)], w_v)

        tail = d % 16
        lane = lax.iota(jnp.int32, 16)
        tmask = lane < tail
        tcol = (d - tail) + lane

        zeros16 = jnp.zeros((16,), jnp.float32)
        for r in range(zb):
            for j in range(d // 16):
                zb_v[r, pl.ds(j * 16, 16)] = zeros16
            if tail:
                plsc.store_scatter(zb_v, [jnp.full((16,), r, jnp.int32), tcol],
                                   zeros16, mask=tmask)
        acc0 = sid * rpt

        for fh in range(nf):
            m_f = m_hbm.at[fh]

            for i in range(rpt // zb):
                pltpu.sync_copy(zb_v, acc_sh.at[pl.ds(acc0 + i * zb, zb)])
            plsc.subcore_barrier()

            def gather_start(c, buf):
                pltpu.async_copy(m_f.at[src_v.at[c]], rows_v.at[buf], gsem)

            def gather_wait(c, buf):
                pltpu.make_async_copy(m_f.at[src_v.at[c]], rows_v.at[buf],
                                      gsem).wait()

            def multiply(c, buf):
                c16 = jnp.full((16,), c, jnp.int32)

                @plsc.parallel_loop(0, ch, unroll=4)
                def edge(e):
                    e16 = jnp.full((16,), e, jnp.int32)
                    ws = plsc.load_gather(w_v, [c16, e16])
                    for j in range(d // 16):
                        sl = pl.ds(j * 16, 16)
                        srows_v[buf, e, sl] = rows_v[buf, e, sl] * ws
                    if tail:
                        tv = plsc.load_gather(rows_v.at[buf], [e16, tcol],
                                              mask=tmask)
                        plsc.store_scatter(srows_v.at[buf], [e16, tcol],
                                           tv * ws, mask=tmask)

            def scatter_start(c, buf):
                pltpu.async_copy(srows_v.at[buf], acc_sh.at[dst_v.at[c]],
                                 ssem, add=True)

            def scatter_wait(c, buf):
                pltpu.make_async_copy(srows_v.at[buf], acc_sh.at[dst_v.at[c]],
                                      ssem).wait()

            gather_start(0, 0)

            def pair(g, _):
                c = g * 2
                gather_start(c + 1, 1)
                gather_wait(c, 0)

                @pl.when(g > 0)
                def _():
                    scatter_wait(c - 2, 0)
                multiply(c, 0)
                scatter_start(c, 0)
                gather_start(c + 2, 0)
                gather_wait(c + 1, 1)

                @pl.when(g > 0)
                def _():
                    scatter_wait(c - 1, 1)
                multiply(c + 1, 1)
                scatter_start(c + 1, 1)
                return 0

            lax.fori_loop(0, (cpw - 1) // 2, pair, 0)
            gather_wait(cpw - 1, 0)
            scatter_wait(cpw - 3, 0)
            multiply(cpw - 1, 0)
            scatter_start(cpw - 1, 0)
            scatter_wait(cpw - 2, 1)
            scatter_wait(cpw - 1, 0)

            plsc.subcore_barrier()
            pltpu.sync_copy(acc_sh.at[pl.ds(acc0, rpt)],
                            out_hbm.at[cid, fh, pl.ds(acc0, rpt)])
            plsc.subcore_barrier()

    return spmm(m, src2, dst2, w2)


def _mm1_kernel(a_ref, b_ref, o_ref):
    a = a_ref[...]
    o_ref[0] = jnp.dot(a, b_ref[0], preferred_element_type=jnp.float32)
    o_ref[1] = jnp.dot(a, b_ref[1], preferred_element_type=jnp.float32)


def _matmul1(a, b2, bm=400):
    m, k = a.shape
    n = b2.shape[2]
    return pl.pallas_call(
        _mm1_kernel,
        grid=(m // bm,),
        in_specs=[
            pl.BlockSpec((bm, k), lambda i: (i, 0)),
            pl.BlockSpec((2, k, n), lambda i: (0, 0, 0)),
        ],
        out_specs=pl.BlockSpec((2, bm, n), lambda i: (0, i, 0)),
        out_shape=jax.ShapeDtypeStruct((2, m, n), jnp.float32),
    )(a, b2)


def _fuse2_kernel(p_ref, w_ref, h_ref, o_ref):
    hblk = jnp.concatenate(
        [jnp.maximum(p_ref[0, 0] + p_ref[1, 0], 0.0),
         jnp.maximum(p_ref[0, 1] + p_ref[1, 1], 0.0)], axis=-1)
    h_ref[...] = hblk
    o_ref[...] = jnp.dot(hblk, w_ref[...], preferred_element_type=jnp.float32)


def _fuse2(p1, w2, bm=400):
    _, _, m, k2 = p1.shape
    n = w2.shape[1]
    return pl.pallas_call(
        _fuse2_kernel,
        grid=(m // bm,),
        in_specs=[
            pl.BlockSpec((2, 2, bm, k2), lambda i: (0, 0, i, 0)),
            pl.BlockSpec((2 * k2, n), lambda i: (0, 0)),
        ],
        out_specs=[
            pl.BlockSpec((bm, 2 * k2), lambda i: (i, 0)),
            pl.BlockSpec((bm, n), lambda i: (i, 0)),
        ],
        out_shape=[
            jax.ShapeDtypeStruct((m, 2 * k2), jnp.float32),
            jax.ShapeDtypeStruct((m, n), jnp.float32),
        ],
    )(p1, w2)


def _comb_kernel(n_out, p_ref, o_ref):
    o_ref[...] = (p_ref[0, 0] + p_ref[1, 0])[:, :n_out]


def _combine(p2, n_out, bm=400):
    _, _, m, n = p2.shape
    return pl.pallas_call(
        functools.partial(_comb_kernel, n_out),
        grid=(m // bm,),
        in_specs=[pl.BlockSpec((2, 1, bm, n), lambda i: (0, 0, i, 0))],
        out_specs=pl.BlockSpec((bm, n_out), lambda i: (i, 0)),
        out_shape=jax.ShapeDtypeStruct((m, n_out), jnp.float32),
    )(p2)


def kernel(x, edge_index, edge_weight, W1, W2):
    n = x.shape[0]
    d_hid = W1.shape[1]
    d_out = W2.shape[1]
    src2 = edge_index[0].reshape(-1, _CH)
    dst2 = edge_index[1].reshape(-1, _CH)
    w2e = edge_weight.reshape(-1, _CH)
    w1s = W1.T.reshape(2, d_hid // 2, W1.shape[0]).transpose(0, 2, 1)
    pad = (-d_out) % 16
    w2p = jnp.concatenate(
        [W2, jnp.zeros((W2.shape[0], pad), W2.dtype)], axis=1)

    xw1 = _matmul1(x, w1s)
    p1 = _spmm_sc(xw1, src2, dst2, w2e, n)
    h, hw2 = _fuse2(p1, w2p)
    p2 = _spmm_sc(hw2[None], src2, dst2, w2e, n)
    out = _combine(p2, d_out)
    return (out, h)


# speedup vs baseline: 1.3919x; 1.0194x over previous
import functools

import jax
import jax.numpy as jnp
from jax import lax
from jax.experimental import pallas as pl
from jax.experimental.pallas import tpu as pltpu
from jax.experimental.pallas import tpu_sc as plsc

_NC = 2
_NS = 16
_NW = _NC * _NS
_CH = 80


def _spmm_sc(m, src2, dst2, w2, n_nodes):
    nf, _, d = m.shape
    n_chunks, ch = src2.shape
    cpw = n_chunks // _NW
    rpt = n_nodes // _NS
    zb = 25
    mesh = plsc.VectorSubcoreMesh(core_axis_name="c", subcore_axis_name="s")

    @functools.partial(
        pl.kernel,
        out_type=jax.ShapeDtypeStruct((_NC, nf, n_nodes, d), jnp.float32),
        mesh=mesh,
        scratch_types=[
            pltpu.VMEM((cpw, ch), jnp.int32),
            pltpu.VMEM((cpw, ch), jnp.int32),
            pltpu.VMEM((cpw, ch), jnp.float32),
            pltpu.VMEM((2, ch, d), jnp.float32),
            pltpu.VMEM((2, ch, d), jnp.float32),
            pltpu.VMEM((zb, d), jnp.float32),
            pltpu.VMEM_SHARED((n_nodes, d), jnp.float32),
            pltpu.SemaphoreType.DMA,
            pltpu.SemaphoreType.DMA,
        ],
        compiler_params=pltpu.CompilerParams(use_tc_tiling_on_sc=False,
                                             needs_layout_passes=False),
    )
    def spmm(m_hbm, src_hbm, dst_hbm, w_hbm, out_hbm,
             src_v, dst_v, w_v, rows_v, srows_v, zb_v, acc_sh, gsem, ssem):
        cid = lax.axis_index("c")
        sid = lax.axis_index("s")
        wid = cid * _NS + sid
        row0 = wid * cpw

        pltpu.async_copy(src_hbm.at[pl.ds(row0, cpw)], src_v, gsem)
        pltpu.async_copy(dst_hbm.at[pl.ds(row0, cpw)], dst_v, gsem)
        pltpu.async_copy(w_hbm.at[pl.ds(row0, cpw)], w_v, gsem)

        tail = d % 16
        lane = lax.iota(jnp.int32, 16)
        tmask = lane < tail
        tcol = (d - tail) + lane

        zeros16 = jnp.zeros((16,), jnp.float32)
        for r in range(zb):
            for j in range(d // 16):
                zb_v[r, pl.ds(j * 16, 16)] = zeros16
            if tail:
                plsc.store_scatter(zb_v, [jnp.full((16,), r, jnp.int32), tcol],
                                   zeros16, mask=tmask)
        acc0 = sid * rpt

        for fh in range(nf):
            m_f = m_hbm.at[fh]

            for i in range(rpt // zb):
                pltpu.sync_copy(zb_v, acc_sh.at[pl.ds(acc0 + i * zb, zb)])
            if fh == 0:
                pltpu.make_async_copy(src_hbm.at[pl.ds(row0, cpw)], src_v,
                                      gsem).wait()
                pltpu.make_async_copy(dst_hbm.at[pl.ds(row0, cpw)], dst_v,
                                      gsem).wait()
                pltpu.make_async_copy(w_hbm.at[pl.ds(row0, cpw)], w_v,
                                      gsem).wait()
            plsc.subcore_barrier()

            def gather_start(c, buf):
                pltpu.async_copy(m_f.at[src_v.at[c]], rows_v.at[buf], gsem)

            def gather_wait(c, buf):
                pltpu.make_async_copy(m_f.at[src_v.at[c]], rows_v.at[buf],
                                      gsem).wait()

            def multiply(c, buf):
                c16 = jnp.full((16,), c, jnp.int32)

                @plsc.parallel_loop(0, ch, unroll=4)
                def edge(e):
                    e16 = jnp.full((16,), e, jnp.int32)
                    ws = plsc.load_gather(w_v, [c16, e16])
                    for j in range(d // 16):
                        sl = pl.ds(j * 16, 16)
                        srows_v[buf, e, sl] = rows_v[buf, e, sl] * ws
                    if tail:
                        tv = plsc.load_gather(rows_v.at[buf], [e16, tcol],
                                              mask=tmask)
                        plsc.store_scatter(srows_v.at[buf], [e16, tcol],
                                           tv * ws, mask=tmask)

            def scatter_start(c, buf):
                pltpu.async_copy(srows_v.at[buf], acc_sh.at[dst_v.at[c]],
                                 ssem, add=True)

            def scatter_wait(c, buf):
                pltpu.make_async_copy(srows_v.at[buf], acc_sh.at[dst_v.at[c]],
                                      ssem).wait()

            gather_start(0, 0)

            def pair(g, _):
                c = g * 2
                gather_start(c + 1, 1)
                gather_wait(c, 0)

                @pl.when(g > 0)
                def _():
                    scatter_wait(c - 2, 0)
                multiply(c, 0)
                scatter_start(c, 0)
                gather_start(c + 2, 0)
                gather_wait(c + 1, 1)

                @pl.when(g > 0)
                def _():
                    scatter_wait(c - 1, 1)
                multiply(c + 1, 1)
                scatter_start(c + 1, 1)
                return 0

            lax.fori_loop(0, (cpw - 1) // 2, pair, 0)
            gather_wait(cpw - 1, 0)
            scatter_wait(cpw - 3, 0)
            multiply(cpw - 1, 0)
            scatter_start(cpw - 1, 0)
            scatter_wait(cpw - 2, 1)
            scatter_wait(cpw - 1, 0)

            plsc.subcore_barrier()
            pltpu.sync_copy(acc_sh.at[pl.ds(acc0, rpt)],
                            out_hbm.at[cid, fh, pl.ds(acc0, rpt)])
            plsc.subcore_barrier()

    return spmm(m, src2, dst2, w2)


def _mm1_kernel(a_ref, b_ref, o_ref):
    a = a_ref[...]
    o_ref[0] = jnp.dot(a, b_ref[0], preferred_element_type=jnp.float32)
    o_ref[1] = jnp.dot(a, b_ref[1], preferred_element_type=jnp.float32)


def _matmul1(a, b2, bm=400):
    m, k = a.shape
    n = b2.shape[2]
    return pl.pallas_call(
        _mm1_kernel,
        grid=(m // bm,),
        in_specs=[
            pl.BlockSpec((bm, k), lambda i: (i, 0)),
            pl.BlockSpec((2, k, n), lambda i: (0, 0, 0)),
        ],
        out_specs=pl.BlockSpec((2, bm, n), lambda i: (0, i, 0)),
        out_shape=jax.ShapeDtypeStruct((2, m, n), jnp.float32),
    )(a, b2)


def _fuse2_kernel(p_ref, w_ref, h_ref, o_ref):
    hblk = jnp.concatenate(
        [jnp.maximum(p_ref[0, 0] + p_ref[1, 0], 0.0),
         jnp.maximum(p_ref[0, 1] + p_ref[1, 1], 0.0)], axis=-1)
    h_ref[...] = hblk
    o_ref[...] = jnp.dot(hblk, w_ref[...], preferred_element_type=jnp.float32)


def _fuse2(p1, w2, bm=400):
    _, _, m, k2 = p1.shape
    n = w2.shape[1]
    return pl.pallas_call(
        _fuse2_kernel,
        grid=(m // bm,),
        in_specs=[
            pl.BlockSpec((2, 2, bm, k2), lambda i: (0, 0, i, 0)),
            pl.BlockSpec((2 * k2, n), lambda i: (0, 0)),
        ],
        out_specs=[
            pl.BlockSpec((bm, 2 * k2), lambda i: (i, 0)),
            pl.BlockSpec((bm, n), lambda i: (i, 0)),
        ],
        out_shape=[
            jax.ShapeDtypeStruct((m, 2 * k2), jnp.float32),
            jax.ShapeDtypeStruct((m, n), jnp.float32),
        ],
    )(p1, w2)


def _comb_kernel(n_out, p_ref, o_ref):
    o_ref[...] = (p_ref[0, 0] + p_ref[1, 0])[:, :n_out]


def _combine(p2, n_out, bm=400):
    _, _, m, n = p2.shape
    return pl.pallas_call(
        functools.partial(_comb_kernel, n_out),
        grid=(m // bm,),
        in_specs=[pl.BlockSpec((2, 1, bm, n), lambda i: (0, 0, i, 0))],
        out_specs=pl.BlockSpec((bm, n_out), lambda i: (i, 0)),
        out_shape=jax.ShapeDtypeStruct((m, n_out), jnp.float32),
    )(p2)


def kernel(x, edge_index, edge_weight, W1, W2):
    n = x.shape[0]
    d_hid = W1.shape[1]
    d_out = W2.shape[1]
    src2 = edge_index[0].reshape(-1, _CH)
    dst2 = edge_index[1].reshape(-1, _CH)
    w2e = edge_weight.reshape(-1, _CH)
    w1s = W1.T.reshape(2, d_hid // 2, W1.shape[0]).transpose(0, 2, 1)
    pad = (-d_out) % 16
    w2p = jnp.concatenate(
        [W2, jnp.zeros((W2.shape[0], pad), W2.dtype)], axis=1)

    xw1 = _matmul1(x, w1s)
    p1 = _spmm_sc(xw1, src2, dst2, w2e, n)
    h, hw2 = _fuse2(p1, w2p)
    p2 = _spmm_sc(hw2[None], src2, dst2, w2e, n)
    out = _combine(p2, d_out)
    return (out, h)
